# baseline (speedup 1.0000x reference)
import functools

import jax
import jax.numpy as jnp
from jax import lax
from jax.experimental import pallas as pl
from jax.experimental.pallas import tpu as pltpu
from jax.experimental.pallas import tpu_sc as plsc

B, S, D, K = 4, 2048, 256, 16
BS = B * S
BSK = BS * K

RB = 256


def _topk_body(boff, fall_ref, fblk_ref, idx_ref):
    b = pl.program_id(0) + boff
    fall = fall_ref[0]
    fblk = fblk_ref[0]
    colsq = jnp.sum(fall * fall, axis=1, keepdims=True)
    e = lax.dot_general(fblk.astype(jnp.bfloat16), fall.astype(jnp.bfloat16),
                        (((1,), (1,)), ((), ())),
                        preferred_element_type=jnp.float32)
    d = -2.0 * e + jnp.transpose(colsq)
    iota = lax.broadcasted_iota(jnp.int32, (RB, S), 1)
    big_i = jnp.int32(S)
    picks = []
    for _ in range(K):
        m = jnp.min(d, axis=1, keepdims=True)
        eqm = d == m
        amin = jnp.min(jnp.where(eqm, iota, big_i), axis=1, keepdims=True)
        picks.append(amin)
        d = jnp.where(eqm, jnp.inf, d)
    idx = jnp.concatenate(picks, axis=1)
    idx_ref[0] = idx + b * S


def _topk(features, boff):
    nb = features.shape[0]
    return pl.pallas_call(
        functools.partial(_topk_body, boff),
        grid=(nb, S // RB),
        in_specs=[
            pl.BlockSpec((1, S, D), lambda b, i: (b, 0, 0)),
            pl.BlockSpec((1, RB, D), lambda b, i: (b, i, 0)),
        ],
        out_specs=pl.BlockSpec((1, RB, K), lambda b, i: (b, i, 0)),
        out_shape=jax.ShapeDtypeStruct((nb, S, K), jnp.int32),
    )(features, features)


RM = 512
H = D // 2


def _pack_bf16_pair(x):
    bits = lax.bitcast_convert_type(
        x.astype(jnp.bfloat16).astype(jnp.float32), jnp.uint32)
    lo = bits[:, :H] >> jnp.uint32(16)
    hi = bits[:, H:] & jnp.uint32(0xFFFF0000)
    return hi | lo


def _unpack_bf16_pair(w):
    lo = lax.bitcast_convert_type(w << jnp.uint32(16), jnp.float32)
    hi = lax.bitcast_convert_type(w & jnp.uint32(0xFFFF0000), jnp.float32)
    return jnp.concatenate([lo, hi], axis=1)


def _proj_body(f_ref, w1_ref, b1_ref, wq_ref, wk_ref, wv_ref, wg1_ref, bg1_ref,
               qgb_ref, tab_ref):
    f = f_ref[...]
    h = jnp.dot(f, w1_ref[...], preferred_element_type=jnp.float32) + b1_ref[...]
    wg1 = wg1_ref[...]
    q = jnp.dot(h, wq_ref[...], preferred_element_type=jnp.float32)
    qgb_ref[...] = (jnp.dot(q, wg1, preferred_element_type=jnp.float32)
                    + bg1_ref[...])
    hk = jnp.dot(h, wk_ref[...], preferred_element_type=jnp.float32)
    kg = jnp.dot(hk, wg1, preferred_element_type=jnp.float32)
    hv = jnp.dot(h, wv_ref[...], preferred_element_type=jnp.float32)
    tab_ref[:, :H] = _pack_bf16_pair(kg)
    tab_ref[:, H:] = _pack_bf16_pair(hv)


def _proj(f2, w1, b1, wq, wk, wv, wg1, bg1):
    wspec = pl.BlockSpec((D, D), lambda i: (0, 0))
    bspec = pl.BlockSpec((1, D), lambda i: (0, 0))
    return pl.pallas_call(
        _proj_body,
        grid=(BS // RM,),
        in_specs=[pl.BlockSpec((RM, D), lambda i: (i, 0)),
                  wspec, bspec, wspec, wspec, wspec, wspec, bspec],
        out_specs=[pl.BlockSpec((RM, D), lambda i: (i, 0)),
                   pl.BlockSpec((RM, D), lambda i: (i, 0))],
        out_shape=[jax.ShapeDtypeStruct((BS, D), jnp.float32),
                   jax.ShapeDtypeStruct((BS, D), jnp.uint32)],
    )(f2, w1, b1, wq, wk, wv, wg1, bg1)


_NW = 32
_C = 128


def _sc_gather(idxf, table):
    n = idxf.shape[0]
    bw = n // _NW
    nch = bw // _C
    mesh = plsc.VectorSubcoreMesh(core_axis_name="c", subcore_axis_name="s")

    @functools.partial(
        pl.kernel,
        out_type=jax.ShapeDtypeStruct((n, D), jnp.uint32),
        mesh=mesh,
        scratch_types=[
            pltpu.VMEM((_C,), jnp.int32), pltpu.VMEM((_C,), jnp.int32),
            pltpu.VMEM((_C, D), jnp.uint32), pltpu.VMEM((_C, D), jnp.uint32),
            pltpu.SemaphoreType.DMA, pltpu.SemaphoreType.DMA,
            pltpu.SemaphoreType.DMA, pltpu.SemaphoreType.DMA,
            pltpu.SemaphoreType.DMA, pltpu.SemaphoreType.DMA,
        ],
    )
    def gk(idx_hbm, table_hbm, out_hbm, idx0, idx1, rows0, rows1,
           is0, is1, gs0, gs1, os0, os1):
        wid = lax.axis_index("s") * 2 + lax.axis_index("c")
        base0 = wid * bw
        idx_v = (idx0, idx1)
        rows_v = (rows0, rows1)
        isem = (is0, is1)
        gsem = (gs0, gs1)
        osem = (os0, os1)

        def idx_start(c, b):
            pltpu.async_copy(idx_hbm.at[pl.ds(base0 + c * _C, _C)],
                             idx_v[b], isem[b])

        def idx_wait(b):
            pltpu.make_async_copy(idx_hbm.at[pl.ds(base0, _C)],
                                  idx_v[b], isem[b]).wait()

        def gather_start(b):
            pltpu.async_copy(table_hbm.at[idx_v[b]], rows_v[b], gsem[b])

        def gather_wait(b):
            pltpu.make_async_copy(table_hbm.at[idx_v[b]], rows_v[b],
                                  gsem[b]).wait()

        def store_start(c, b):
            pltpu.async_copy(rows_v[b], out_hbm.at[pl.ds(base0 + c * _C, _C)],
                             osem[b])

        def store_wait(b):
            pltpu.make_async_copy(rows_v[0], out_hbm.at[pl.ds(base0, _C)],
                                  osem[b]).wait()

        idx_start(0, 0)
        idx_start(1, 1)
        idx_wait(0)
        gather_start(0)

        def pair(g, carry):
            c0 = 2 * g
            idx_wait(1)
            gather_wait(0)
            store_start(c0, 0)

            @pl.when(g > 0)
            def _():
                store_wait(1)
            gather_start(1)

            @pl.when(g < nch // 2 - 1)
            def _():
                idx_start(c0 + 2, 0)
            gather_wait(1)
            store_wait(0)
            store_start(c0 + 1, 1)

            @pl.when(g < nch // 2 - 1)
            def _():
                idx_start(c0 + 3, 1)
                idx_wait(0)
                gather_start(0)
            return carry

        lax.fori_loop(0, nch // 2, pair, 0)
        store_wait(1)

    return gk(idxf, table)


RA = 128


def _attn_body(qgb_ref, gath_ref, f_ref, wg2_ref, bg2_ref, w2_ref, b2_ref,
               out_ref):
    qg = qgb_ref[...]
    kg = _unpack_bf16_pair(gath_ref[:, :H]).reshape(RA, K, D)
    hv = _unpack_bf16_pair(gath_ref[:, H:]).reshape(RA, K, D)
    z = qg[:, None, :] - kg
    a = jnp.maximum(z, 0.0).reshape(RA * K, D).astype(jnp.bfloat16)
    g2 = (jnp.dot(a, wg2_ref[...], preferred_element_type=jnp.float32)
          + bg2_ref[...]).reshape(RA, K, D)
    e = jnp.exp(g2 * (1.0 / 16.0))
    att = e / jnp.sum(e, axis=1, keepdims=True)
    o = jnp.sum(att * hv, axis=1).astype(jnp.bfloat16)
    out_ref[...] = (jnp.dot(o, w2_ref[...], preferred_element_type=jnp.float32)
                    + b2_ref[...] + f_ref[...])


def _attn(qgb, gath, f2, wg2, bg2, w2, b2):
    nr = qgb.shape[0]
    wspec = pl.BlockSpec((D, D), lambda i: (0, 0))
    bspec = pl.BlockSpec((1, D), lambda i: (0, 0))
    return pl.pallas_call(
        _attn_body,
        grid=(nr // RA,),
        in_specs=[pl.BlockSpec((RA, D), lambda i: (i, 0)),
                  pl.BlockSpec((RA * K, D), lambda i: (i, 0)),
                  pl.BlockSpec((RA, D), lambda i: (i, 0)),
                  wspec, bspec, wspec, bspec],
        out_specs=pl.BlockSpec((RA, D), lambda i: (i, 0)),
        out_shape=jax.ShapeDtypeStruct((nr, D), jnp.float32),
    )(qgb, gath, f2, wg2, bg2, w2, b2)


def kernel(x, W1, b1, W2, b2, Wg1, bg1, Wg2, bg2, Wq, Wk, Wv):
    pos = x[:, :, :3]
    feats = x[:, :, 3:]
    f2 = feats.reshape(BS, D)
    qgb, table = _proj(f2, W1, b1.reshape(1, D), Wq, Wk, Wv, Wg1,
                       bg1.reshape(1, D))
    wg2b, w2b = Wg2.astype(jnp.bfloat16), W2.astype(jnp.bfloat16)
    bg2r, b2r = bg2.reshape(1, D), b2.reshape(1, D)
    nc = 2
    bc = B // nc
    rows = bc * S
    res_chunks = []
    gaths, gidxs = [], []
    for c in range(nc):
        gidxs.append(_topk(feats[c * bc:(c + 1) * bc], c * bc))
        gaths.append(_sc_gather(gidxs[c].reshape(rows * K), table))
    for c in range(nc):
        lo = c * rows
        res_chunks.append(_attn(qgb[lo:lo + rows], gaths[c],
                                f2[lo:lo + rows], wg2b, bg2r, w2b, b2r))
    res = jnp.concatenate(res_chunks, axis=0)
    return (pos, res.reshape(B, S, D))


# speedup vs baseline: 1.0416x; 1.0416x over previous
import functools

import jax
import jax.numpy as jnp
from jax import lax
from jax.experimental import pallas as pl
from jax.experimental.pallas import tpu as pltpu
from jax.experimental.pallas import tpu_sc as plsc

B, S, D, K = 4, 2048, 256, 16
BS = B * S
BSK = BS * K

RB = 256


def _topk_body(boff, fall_ref, fblk_ref, idx_ref):
    b = pl.program_id(0) + boff
    fall = fall_ref[0]
    fblk = fblk_ref[0]
    colsq = jnp.sum(fall * fall, axis=1, keepdims=True)
    e = lax.dot_general(fblk.astype(jnp.bfloat16), fall.astype(jnp.bfloat16),
                        (((1,), (1,)), ((), ())),
                        preferred_element_type=jnp.float32)
    d = -2.0 * e + jnp.transpose(colsq)
    iota = lax.broadcasted_iota(jnp.int32, (RB, S), 1)
    big_i = jnp.int32(S)
    i = pl.program_id(1)
    rr = lax.broadcasted_iota(jnp.int32, (RB, 1), 0) + i * RB
    picks = [rr]
    d = jnp.where(iota == rr, jnp.inf, d)
    for _ in range(K - 1):
        m = jnp.min(d, axis=1, keepdims=True)
        eqm = d == m
        amin = jnp.min(jnp.where(eqm, iota, big_i), axis=1, keepdims=True)
        picks.append(amin)
        d = jnp.where(eqm, jnp.inf, d)
    idx = jnp.concatenate(picks, axis=1)
    idx_ref[0] = idx + b * S


def _topk(features, boff):
    nb = features.shape[0]
    return pl.pallas_call(
        functools.partial(_topk_body, boff),
        grid=(nb, S // RB),
        in_specs=[
            pl.BlockSpec((1, S, D), lambda b, i: (b, 0, 0)),
            pl.BlockSpec((1, RB, D), lambda b, i: (b, i, 0)),
        ],
        out_specs=pl.BlockSpec((1, RB, K), lambda b, i: (b, i, 0)),
        out_shape=jax.ShapeDtypeStruct((nb, S, K), jnp.int32),
    )(features, features)


RM = 512
H = D // 2


def _pack_bf16_pair(x):
    bits = lax.bitcast_convert_type(
        x.astype(jnp.bfloat16).astype(jnp.float32), jnp.uint32)
    lo = bits[:, :H] >> jnp.uint32(16)
    hi = bits[:, H:] & jnp.uint32(0xFFFF0000)
    return hi | lo


def _unpack_bf16_pair(w):
    lo = lax.bitcast_convert_type(w << jnp.uint32(16), jnp.float32)
    hi = lax.bitcast_convert_type(w & jnp.uint32(0xFFFF0000), jnp.float32)
    return jnp.concatenate([lo, hi], axis=1)


def _proj_body(f_ref, w1_ref, b1_ref, wq_ref, wk_ref, wv_ref, wg1_ref, bg1_ref,
               qgb_ref, tab_ref):
    f = f_ref[...]
    h = jnp.dot(f, w1_ref[...], preferred_element_type=jnp.float32) + b1_ref[...]
    wg1 = wg1_ref[...]
    q = jnp.dot(h, wq_ref[...], preferred_element_type=jnp.float32)
    qgb_ref[...] = (jnp.dot(q, wg1, preferred_element_type=jnp.float32)
                    + bg1_ref[...])
    hk = jnp.dot(h, wk_ref[...], preferred_element_type=jnp.float32)
    kg = jnp.dot(hk, wg1, preferred_element_type=jnp.float32)
    hv = jnp.dot(h, wv_ref[...], preferred_element_type=jnp.float32)
    tab_ref[:, :H] = _pack_bf16_pair(kg)
    tab_ref[:, H:] = _pack_bf16_pair(hv)


def _proj(f2, w1, b1, wq, wk, wv, wg1, bg1):
    wspec = pl.BlockSpec((D, D), lambda i: (0, 0))
    bspec = pl.BlockSpec((1, D), lambda i: (0, 0))
    return pl.pallas_call(
        _proj_body,
        grid=(BS // RM,),
        in_specs=[pl.BlockSpec((RM, D), lambda i: (i, 0)),
                  wspec, bspec, wspec, wspec, wspec, wspec, bspec],
        out_specs=[pl.BlockSpec((RM, D), lambda i: (i, 0)),
                   pl.BlockSpec((RM, D), lambda i: (i, 0))],
        out_shape=[jax.ShapeDtypeStruct((BS, D), jnp.float32),
                   jax.ShapeDtypeStruct((BS, D), jnp.uint32)],
    )(f2, w1, b1, wq, wk, wv, wg1, bg1)


_NW = 32
_C = 64


def _sc_gather(idxf, table):
    n = idxf.shape[0]
    bw = n // _NW
    nch = bw // _C
    mesh = plsc.VectorSubcoreMesh(core_axis_name="c", subcore_axis_name="s")

    @functools.partial(
        pl.kernel,
        out_type=jax.ShapeDtypeStruct((n, D), jnp.uint32),
        mesh=mesh,
        scratch_types=[
            pltpu.VMEM((_C,), jnp.int32),
            pltpu.VMEM((_C, D), jnp.uint32),
            pltpu.SemaphoreType.DMA,
        ],
    )
    def gk(idx_hbm, table_hbm, out_hbm, idx_v, rows_v, sem):
        wid = lax.axis_index("s") * 2 + lax.axis_index("c")

        def chunk(i, carry):
            base = wid * bw + i * _C
            pltpu.sync_copy(idx_hbm.at[pl.ds(base, _C)], idx_v)
            pltpu.async_copy(table_hbm.at[idx_v], rows_v, sem).wait()
            pltpu.sync_copy(rows_v, out_hbm.at[pl.ds(base, _C)])
            return carry

        lax.fori_loop(0, nch, chunk, 0)

    return gk(idxf, table)


RA = 128


def _attn_body(qgb_ref, gath_ref, f_ref, wg2_ref, bg2_ref, w2_ref, b2_ref,
               out_ref):
    qg = qgb_ref[...]
    kg = _unpack_bf16_pair(gath_ref[:, :H]).reshape(RA, K, D)
    hv = _unpack_bf16_pair(gath_ref[:, H:]).reshape(RA, K, D)
    z = qg[:, None, :] - kg
    a = jnp.maximum(z, 0.0).reshape(RA * K, D).astype(jnp.bfloat16)
    g2 = (jnp.dot(a, wg2_ref[...], preferred_element_type=jnp.float32)
          + bg2_ref[...]).reshape(RA, K, D)
    e = jnp.exp(g2 * (1.0 / 16.0))
    att = e / jnp.sum(e, axis=1, keepdims=True)
    o = jnp.sum(att * hv, axis=1).astype(jnp.bfloat16)
    out_ref[...] = (jnp.dot(o, w2_ref[...], preferred_element_type=jnp.float32)
                    + b2_ref[...] + f_ref[...])


def _attn(qgb, gath, f2, wg2, bg2, w2, b2):
    nr = qgb.shape[0]
    wspec = pl.BlockSpec((D, D), lambda i: (0, 0))
    bspec = pl.BlockSpec((1, D), lambda i: (0, 0))
    return pl.pallas_call(
        _attn_body,
        grid=(nr // RA,),
        in_specs=[pl.BlockSpec((RA, D), lambda i: (i, 0)),
                  pl.BlockSpec((RA * K, D), lambda i: (i, 0)),
                  pl.BlockSpec((RA, D), lambda i: (i, 0)),
                  wspec, bspec, wspec, bspec],
        out_specs=pl.BlockSpec((RA, D), lambda i: (i, 0)),
        out_shape=jax.ShapeDtypeStruct((nr, D), jnp.float32),
    )(qgb, gath, f2, wg2, bg2, w2, b2)


def kernel(x, W1, b1, W2, b2, Wg1, bg1, Wg2, bg2, Wq, Wk, Wv):
    pos = x[:, :, :3]
    feats = x[:, :, 3:]
    f2 = feats.reshape(BS, D)
    qgb, table = _proj(f2, W1, b1.reshape(1, D), Wq, Wk, Wv, Wg1,
                       bg1.reshape(1, D))
    wg2b, w2b = Wg2.astype(jnp.bfloat16), W2.astype(jnp.bfloat16)
    bg2r, b2r = bg2.reshape(1, D), b2.reshape(1, D)
    nc = 2
    bc = B // nc
    rows = bc * S
    res_chunks = []
    gaths, gidxs = [], []
    for c in range(nc):
        gidxs.append(_topk(feats[c * bc:(c + 1) * bc], c * bc))
        gaths.append(_sc_gather(gidxs[c].reshape(rows * K), table))
    for c in range(nc):
        lo = c * rows
        res_chunks.append(_attn(qgb[lo:lo + rows], gaths[c],
                                f2[lo:lo + rows], wg2b, bg2r, w2b, b2r))
    res = jnp.concatenate(res_chunks, axis=0)
    return (pos, res.reshape(B, S, D))
